# Initial kernel scaffold; baseline (speedup 1.0000x reference)
import jax
import jax.numpy as jnp
from jax.experimental import pallas as pl


def kernel(num_x, cat_x, tables, W1, b1, W2, b2, W3, b3):
    raise NotImplementedError("write your pallas kernel here")


# speedup vs baseline: 2.1814x; 2.1814x over previous
import functools

import jax
import jax.numpy as jnp
from jax import lax
from jax.experimental import pallas as pl
from jax.experimental.pallas import tpu as pltpu
from jax.experimental.pallas import tpu_sc as plsc

_B = 16384
_NUM = 13
_F = 26
_V = 100000
_D = 16
_H1 = 256
_H2 = 128
_C = 10

_NW = 32
_ROWS = _B * _F
_RPW = _ROWS // _NW
_CH = 3328
_NCHUNK = _RPW // _CH


def _make_gather():
    mesh = plsc.VectorSubcoreMesh(core_axis_name="c", subcore_axis_name="s")

    @functools.partial(
        pl.kernel,
        mesh=mesh,
        out_type=jax.ShapeDtypeStruct((_ROWS, _D), jnp.float32),
        scratch_types=[
            pltpu.VMEM((_CH,), jnp.int32),
            pltpu.VMEM((_CH, _D), jnp.float32),
            pltpu.SemaphoreType.DMA,
        ],
        compiler_params=pltpu.CompilerParams(use_tc_tiling_on_sc=False),
    )
    def gather_k(table_hbm, idx_hbm, out_hbm, idx_v, rows_v, sem):
        wid = lax.axis_index("s") * 2 + lax.axis_index("c")
        base = wid * _RPW
        for t in range(_NCHUNK):
            off = base + t * _CH
            pltpu.sync_copy(idx_hbm.at[pl.ds(off, _CH)], idx_v)
            pltpu.async_copy(table_hbm.at[idx_v], rows_v, sem).wait()
            pltpu.sync_copy(rows_v, out_hbm.at[pl.ds(off, _CH)])

    return gather_k


_gather = _make_gather()

_BM = 1024


def _mlp_body(num_ref, emb_ref, w1n_ref, w1e_ref, b1_ref, w2_ref, b2_ref,
              w3_ref, b3_ref, out_ref):
    h1 = jnp.dot(emb_ref[...], w1e_ref[...], preferred_element_type=jnp.float32)
    h1 += jnp.dot(num_ref[...], w1n_ref[...], preferred_element_type=jnp.float32)
    h1 = jnp.maximum(h1 + b1_ref[...], 0.0)
    h2 = jnp.dot(h1, w2_ref[...], preferred_element_type=jnp.float32)
    h2 = jnp.maximum(h2 + b2_ref[...], 0.0)
    out_ref[...] = (
        jnp.dot(h2, w3_ref[...], preferred_element_type=jnp.float32) + b3_ref[...]
    )


def _mlp(num_x, emb, W1n, W1e, b1, W2, b2, W3, b3):
    full = lambda shape: pl.BlockSpec(shape, lambda i: (0, 0))
    return pl.pallas_call(
        _mlp_body,
        grid=(_B // _BM,),
        in_specs=[
            pl.BlockSpec((_BM, _NUM), lambda i: (i, 0)),
            pl.BlockSpec((_BM, _F * _D), lambda i: (i, 0)),
            full((_NUM, _H1)),
            full((_F * _D, _H1)),
            full((1, _H1)),
            full((_H1, _H2)),
            full((1, _H2)),
            full((_H2, _C)),
            full((1, _C)),
        ],
        out_specs=pl.BlockSpec((_BM, _C), lambda i: (i, 0)),
        out_shape=jax.ShapeDtypeStruct((_B, _C), jnp.float32),
    )(num_x, emb, W1n, W1e, b1, W2, b2, W3, b3)


def kernel(num_x, cat_x, tables, W1, b1, W2, b2, W3, b3):
    flat_idx = (cat_x + (jnp.arange(_F, dtype=jnp.int32) * _V)[None, :]).reshape(-1)
    table2d = tables.reshape(_F * _V, _D)
    rows = _gather(table2d, flat_idx)
    emb = rows.reshape(_B, _F * _D)
    return _mlp(num_x, emb, W1[:_NUM], W1[_NUM:], b1.reshape(1, _H1),
                W2, b2.reshape(1, _H2), W3, b3.reshape(1, _C))


# speedup vs baseline: 11.0741x; 5.0765x over previous
import functools

import jax
import jax.numpy as jnp
from jax import lax
from jax.experimental import pallas as pl
from jax.experimental.pallas import tpu as pltpu
from jax.experimental.pallas import tpu_sc as plsc

_B = 16384
_NUM = 13
_F = 26
_V = 100000
_D = 16
_H1 = 256
_H2 = 128
_C = 10

_NW = 32
_NTASK = _F * _D
_TPW = _NTASK // _NW
_CHB = 8192
_NCB = _B // _CHB


def _make_gather():
    mesh = plsc.VectorSubcoreMesh(core_axis_name="c", subcore_axis_name="s")

    @functools.partial(
        pl.kernel,
        mesh=mesh,
        out_type=jax.ShapeDtypeStruct((_F, _D, _B), jnp.float32),
        scratch_types=[
            pltpu.VMEM((_V,), jnp.float32),
            pltpu.VMEM((_CHB,), jnp.int32),
            pltpu.VMEM((_CHB,), jnp.float32),
        ],
        compiler_params=pltpu.CompilerParams(needs_layout_passes=False),
    )
    def gather_k(tabT_hbm, catT_hbm, out_hbm, row_v, idx_v, val_v):
        wid = lax.axis_index("s") * 2 + lax.axis_index("c")

        def do_block(_):
            def body(i, carry):
                sl = pl.ds(i * 16, 16)
                val_v[sl] = plsc.load_gather(row_v, [idx_v[sl]])
                return carry
            lax.fori_loop(0, _CHB // 16, body, 0)

        for t in range(_TPW):
            task = wid * _TPW + t
            f = task // _D
            d = task % _D
            pltpu.sync_copy(tabT_hbm.at[f, d], row_v)
            for cb in range(_NCB):
                bsl = pl.ds(cb * _CHB, _CHB)
                pltpu.sync_copy(catT_hbm.at[f, bsl], idx_v)
                do_block(None)
                pltpu.sync_copy(val_v, out_hbm.at[f, d, bsl])

    return gather_k


_gather = _make_gather()

_BN = 1024


def _mlp_body(numT_ref, embT_ref, w1nT_ref, w1eT_ref, b1_ref, w2T_ref,
              b2_ref, w3T_ref, b3_ref, outT_ref):
    h1 = jnp.dot(w1eT_ref[...], embT_ref[...], preferred_element_type=jnp.float32)
    h1 += jnp.dot(w1nT_ref[...], numT_ref[...], preferred_element_type=jnp.float32)
    h1 = jnp.maximum(h1 + b1_ref[...], 0.0)
    h2 = jnp.dot(w2T_ref[...], h1, preferred_element_type=jnp.float32)
    h2 = jnp.maximum(h2 + b2_ref[...], 0.0)
    outT_ref[...] = (
        jnp.dot(w3T_ref[...], h2, preferred_element_type=jnp.float32) + b3_ref[...]
    )


def _mlp(numT, embT, W1nT, W1eT, b1, W2T, b2, W3T, b3):
    full = lambda shape: pl.BlockSpec(shape, lambda i: (0, 0))
    return pl.pallas_call(
        _mlp_body,
        grid=(_B // _BN,),
        in_specs=[
            pl.BlockSpec((_NUM, _BN), lambda i: (0, i)),
            pl.BlockSpec((_F * _D, _BN), lambda i: (0, i)),
            full((_H1, _NUM)),
            full((_H1, _F * _D)),
            full((_H1, 1)),
            full((_H2, _H1)),
            full((_H2, 1)),
            full((_C, _H2)),
            full((_C, 1)),
        ],
        out_specs=pl.BlockSpec((_C, _BN), lambda i: (0, i)),
        out_shape=jax.ShapeDtypeStruct((_C, _B), jnp.float32),
    )(numT, embT, W1nT, W1eT, b1, W2T, b2, W3T, b3)


def kernel(num_x, cat_x, tables, W1, b1, W2, b2, W3, b3):
    tablesT = jnp.transpose(tables, (0, 2, 1))
    catT = cat_x.T
    numT = num_x.T
    embT3 = _gather(tablesT, catT)
    embT = embT3.reshape(_F * _D, _B)
    outT = _mlp(numT, embT,
                W1[:_NUM].T, W1[_NUM:].T, b1.reshape(_H1, 1),
                W2.T, b2.reshape(_H2, 1), W3.T, b3.reshape(_C, 1))
    return outT.T
